# baseline (speedup 1.0000x reference)
import functools

import jax
import jax.numpy as jnp
from jax.experimental import pallas as pl
from jax.experimental.pallas import tpu as pltpu
from jax.experimental.pallas import tpu_sc as plsc

C = 1024
K = 9
D = C * K
R = 512
STEPS = D // R
SIZE = 512

NW = 32
SC_START = 2304
SC_PER_W = (D - SC_START) // NW
CHUNK = 24
NCHUNK = SC_PER_W // CHUNK
R2 = SC_START // STEPS

_HI = jax.lax.Precision.HIGHEST


def _sc_body(v2_hbm, out_hbm, buf0, buf1, acc, sem0, sem1):
    cid = jax.lax.axis_index("c")
    sid = jax.lax.axis_index("s")
    wid = sid * 2 + cid
    base = SC_START + wid * SC_PER_W

    for j in range(C // 16):
        acc[pl.ds(16 * j, 16)] = jnp.zeros((16,), jnp.float32)

    bufs = (buf0, buf1)
    sems = (sem0, sem1)
    handles = {0: pltpu.async_copy(
        v2_hbm.at[pl.ds(base, CHUNK)], bufs[0], sems[0])}
    for k in range(NCHUNK):
        if k + 1 < NCHUNK:
            handles[k + 1] = pltpu.async_copy(
                v2_hbm.at[pl.ds(base + (k + 1) * CHUNK, CHUNK)],
                bufs[(k + 1) % 2], sems[(k + 1) % 2])
        handles[k].wait()
        buf = bufs[k % 2]

        def rbody(r, carry, buf=buf):
            for j in range(C // 16):
                plsc.addupdate(acc.at[pl.ds(16 * j, 16)],
                               jnp.abs(buf[r, pl.ds(16 * j, 16)]))
            return carry

        jax.lax.fori_loop(0, CHUNK, rbody, 0)

    pltpu.sync_copy(acc, out_hbm.at[wid])


_sc_partial = functools.partial(
    pl.kernel,
    out_type=jax.ShapeDtypeStruct((NW, C), jnp.float32),
    mesh=plsc.VectorSubcoreMesh(core_axis_name="c", subcore_axis_name="s"),
    scratch_types=[
        pltpu.VMEM((CHUNK, C), jnp.float32),
        pltpu.VMEM((CHUNK, C), jnp.float32),
        pltpu.VMEM((C,), jnp.float32),
        pltpu.SemaphoreType.DMA,
        pltpu.SemaphoreType.DMA,
    ],
)(_sc_body)


def _reduce_body(v1_ref, v2_ref, imp1_ref, cs2_ref, acc_ref):
    i = pl.program_id(0)
    half = (i % 2) * R

    rows = jnp.sum(jnp.abs(v1_ref[...]), axis=1, keepdims=True)

    @pl.when(i < 2)
    def _():
        acc_ref[pl.ds(half, R), :] = rows

    @pl.when(i >= 2)
    def _():
        acc_ref[pl.ds(half, R), :] = acc_ref[pl.ds(half, R), :] + rows

    colpart = jnp.sum(jnp.abs(v2_ref[...]), axis=0, keepdims=True)

    @pl.when(i == 0)
    def _():
        cs2_ref[...] = colpart

    @pl.when(i > 0)
    def _():
        cs2_ref[...] = cs2_ref[...] + colpart

    @pl.when(i == STEPS - 1)
    def _():
        imp1_ref[...] = acc_ref[...]


def _select_body(imp1_ref, cs2_ref, scp_ref, out_ref):
    imp2_row = cs2_ref[...] + jnp.sum(scp_ref[...], axis=0, keepdims=True)
    imp1_col = imp1_ref[...]
    eye = (jax.lax.broadcasted_iota(jnp.int32, (C, C), 0)
           == jax.lax.broadcasted_iota(jnp.int32, (C, C), 1)
           ).astype(jnp.float32)
    imp1_row = jax.lax.dot_general(
        imp1_col, eye, (((0,), (0,)), ((), ())), precision=_HI)
    imp2_col = jax.lax.dot_general(
        eye, imp2_row, (((1,), (1,)), ((), ())), precision=_HI)
    imp_col = imp1_col + imp2_col
    imp_row = imp1_row + imp2_row

    src_i = jax.lax.broadcasted_iota(jnp.int32, (C, C), 1)
    tgt_i = jax.lax.broadcasted_iota(jnp.int32, (C, C), 0)
    sel = (imp_row < imp_col) | ((imp_row == imp_col) & (src_i < tgt_i))
    rank = jnp.sum(jnp.where(sel, 1.0, 0.0), axis=1, keepdims=True)
    maskf = jnp.where(rank < float(SIZE), 1.0, 0.0)

    lower = jnp.where(src_i < tgt_i, 1.0, 0.0).astype(jnp.bfloat16)
    pos = jax.lax.dot_general(
        lower, maskf.astype(jnp.bfloat16), (((1,), (0,)), ((), ())),
        preferred_element_type=jnp.float32)

    slot = jax.lax.broadcasted_iota(jnp.int32, (C, SIZE), 1).astype(jnp.float32)
    w = maskf * jnp.where(pos == slot, 1.0, 0.0)
    lin = (-1.0 + jax.lax.broadcasted_iota(jnp.int32, (C, 1), 0)
           .astype(jnp.float32) * (2.0 / float(C - 1)))
    out_ref[...] = jnp.sum(w * lin, axis=0, keepdims=True)


@jax.jit
def _run(w1, w2):
    v1 = jnp.transpose(w1, (2, 3, 0, 1)).reshape(D, C)
    v2 = jnp.transpose(w2, (2, 3, 0, 1)).reshape(D, C)

    scpart = _sc_partial(v2)

    imp1, cs2 = pl.pallas_call(
        _reduce_body,
        grid=(STEPS,),
        in_specs=[
            pl.BlockSpec((R, C), lambda i: (i, 0)),
            pl.BlockSpec((R2, C), lambda i: (i, 0)),
        ],
        out_specs=[
            pl.BlockSpec((C, 1), lambda i: (0, 0)),
            pl.BlockSpec((1, C), lambda i: (0, 0)),
        ],
        out_shape=[
            jax.ShapeDtypeStruct((C, 1), jnp.float32),
            jax.ShapeDtypeStruct((1, C), jnp.float32),
        ],
        scratch_shapes=[pltpu.VMEM((C, 1), jnp.float32)],
        compiler_params=pltpu.CompilerParams(
            dimension_semantics=("arbitrary",),
        ),
    )(v1, v2)

    out = pl.pallas_call(
        _select_body,
        out_shape=jax.ShapeDtypeStruct((1, SIZE), jnp.float32),
    )(imp1, cs2, scpart)
    return out


def kernel(w1, w2, size):
    return _run(w1, w2).reshape(SIZE) + size * 0


# speedup vs baseline: 1.8160x; 1.8160x over previous
import functools

import jax
import jax.numpy as jnp
from jax.experimental import pallas as pl
from jax.experimental.pallas import tpu as pltpu
from jax.experimental.pallas import tpu_sc as plsc

C = 1024
K = 9
D = C * K
R = 512
STEPS = D // R
SIZE = 512

NW = 32
SC_START = 2304
SC_PER_W = (D - SC_START) // NW
CHUNK = 24
NCHUNK = SC_PER_W // CHUNK
R2 = SC_START // STEPS

_HI = jax.lax.Precision.HIGHEST


def _sc_body(v2_hbm, out_hbm, buf0, buf1, acc, sem0, sem1):
    cid = jax.lax.axis_index("c")
    sid = jax.lax.axis_index("s")
    wid = sid * 2 + cid
    base = SC_START + wid * SC_PER_W

    for j in range(C // 16):
        acc[pl.ds(16 * j, 16)] = jnp.zeros((16,), jnp.float32)

    bufs = (buf0, buf1)
    sems = (sem0, sem1)
    handles = {0: pltpu.async_copy(
        v2_hbm.at[pl.ds(base, CHUNK)], bufs[0], sems[0])}
    for k in range(NCHUNK):
        if k + 1 < NCHUNK:
            handles[k + 1] = pltpu.async_copy(
                v2_hbm.at[pl.ds(base + (k + 1) * CHUNK, CHUNK)],
                bufs[(k + 1) % 2], sems[(k + 1) % 2])
        handles[k].wait()
        buf = bufs[k % 2]

        for jb in range(C // 16 // 8):
            accs = tuple(acc[pl.ds(16 * (8 * jb + j), 16)] for j in range(8))

            def rbody(r, accs, buf=buf, jb=jb):
                return tuple(
                    a + jnp.abs(buf[r, pl.ds(16 * (8 * jb + j), 16)])
                    for j, a in enumerate(accs))

            accs = jax.lax.fori_loop(0, CHUNK, rbody, accs)
            for j in range(8):
                acc[pl.ds(16 * (8 * jb + j), 16)] = accs[j]

    pltpu.sync_copy(acc, out_hbm.at[wid])


_sc_partial = functools.partial(
    pl.kernel,
    out_type=jax.ShapeDtypeStruct((NW, C), jnp.float32),
    mesh=plsc.VectorSubcoreMesh(core_axis_name="c", subcore_axis_name="s"),
    scratch_types=[
        pltpu.VMEM((CHUNK, C), jnp.float32),
        pltpu.VMEM((CHUNK, C), jnp.float32),
        pltpu.VMEM((C,), jnp.float32),
        pltpu.SemaphoreType.DMA,
        pltpu.SemaphoreType.DMA,
    ],
)(_sc_body)


def _reduce_body(v1_ref, v2_ref, imp1_ref, cs2_ref, acc_ref):
    i = pl.program_id(0)
    half = (i % 2) * R

    rows = jnp.sum(jnp.abs(v1_ref[...]), axis=1, keepdims=True)

    @pl.when(i < 2)
    def _():
        acc_ref[pl.ds(half, R), :] = rows

    @pl.when(i >= 2)
    def _():
        acc_ref[pl.ds(half, R), :] = acc_ref[pl.ds(half, R), :] + rows

    colpart = jnp.sum(jnp.abs(v2_ref[...]), axis=0, keepdims=True)

    @pl.when(i == 0)
    def _():
        cs2_ref[...] = colpart

    @pl.when(i > 0)
    def _():
        cs2_ref[...] = cs2_ref[...] + colpart

    @pl.when(i == STEPS - 1)
    def _():
        imp1_ref[...] = acc_ref[...]


def _select_body(imp1_ref, cs2_ref, scp_ref, out_ref):
    imp2_row = cs2_ref[...] + jnp.sum(scp_ref[...], axis=0, keepdims=True)
    imp1_col = imp1_ref[...]
    eye = (jax.lax.broadcasted_iota(jnp.int32, (C, C), 0)
           == jax.lax.broadcasted_iota(jnp.int32, (C, C), 1)
           ).astype(jnp.float32)
    imp1_row = jax.lax.dot_general(
        imp1_col, eye, (((0,), (0,)), ((), ())), precision=_HI)
    imp2_col = jax.lax.dot_general(
        eye, imp2_row, (((1,), (1,)), ((), ())), precision=_HI)
    imp_col = imp1_col + imp2_col
    imp_row = imp1_row + imp2_row

    src_i = jax.lax.broadcasted_iota(jnp.int32, (C, C), 1)
    tgt_i = jax.lax.broadcasted_iota(jnp.int32, (C, C), 0)
    sel = (imp_row < imp_col) | ((imp_row == imp_col) & (src_i < tgt_i))
    rank = jnp.sum(jnp.where(sel, 1.0, 0.0), axis=1, keepdims=True)
    maskf = jnp.where(rank < float(SIZE), 1.0, 0.0)

    lower = jnp.where(src_i < tgt_i, 1.0, 0.0).astype(jnp.bfloat16)
    pos = jax.lax.dot_general(
        lower, maskf.astype(jnp.bfloat16), (((1,), (0,)), ((), ())),
        preferred_element_type=jnp.float32)

    slot = jax.lax.broadcasted_iota(jnp.int32, (C, SIZE), 1).astype(jnp.float32)
    w = maskf * jnp.where(pos == slot, 1.0, 0.0)
    lin = (-1.0 + jax.lax.broadcasted_iota(jnp.int32, (C, 1), 0)
           .astype(jnp.float32) * (2.0 / float(C - 1)))
    out_ref[...] = jnp.sum(w * lin, axis=0, keepdims=True)


@jax.jit
def _run(w1, w2):
    v1 = jnp.transpose(w1, (2, 3, 0, 1)).reshape(D, C)
    v2 = jnp.transpose(w2, (2, 3, 0, 1)).reshape(D, C)

    scpart = _sc_partial(v2)

    imp1, cs2 = pl.pallas_call(
        _reduce_body,
        grid=(STEPS,),
        in_specs=[
            pl.BlockSpec((R, C), lambda i: (i, 0)),
            pl.BlockSpec((R2, C), lambda i: (i, 0)),
        ],
        out_specs=[
            pl.BlockSpec((C, 1), lambda i: (0, 0)),
            pl.BlockSpec((1, C), lambda i: (0, 0)),
        ],
        out_shape=[
            jax.ShapeDtypeStruct((C, 1), jnp.float32),
            jax.ShapeDtypeStruct((1, C), jnp.float32),
        ],
        scratch_shapes=[pltpu.VMEM((C, 1), jnp.float32)],
        compiler_params=pltpu.CompilerParams(
            dimension_semantics=("arbitrary",),
        ),
    )(v1, v2)

    out = pl.pallas_call(
        _select_body,
        out_shape=jax.ShapeDtypeStruct((1, SIZE), jnp.float32),
    )(imp1, cs2, scpart)
    return out


def kernel(w1, w2, size):
    return _run(w1, w2).reshape(SIZE) + size * 0


# speedup vs baseline: 2.0028x; 1.1029x over previous
import functools

import jax
import jax.numpy as jnp
from jax.experimental import pallas as pl
from jax.experimental.pallas import tpu as pltpu
from jax.experimental.pallas import tpu_sc as plsc

C = 1024
K = 9
D = C * K
R = 512
STEPS = D // R
SIZE = 512

NW = 32
SC_START = 6912
SC_PER_W = (D - SC_START) // NW
CHUNK = 24
NCHUNK = SC_PER_W // CHUNK
RUNROLL = 6
R2 = SC_START // STEPS

_HI = jax.lax.Precision.HIGHEST


def _sc_body(v2_hbm, out_hbm, buf0, buf1, buf2, acc, sem0, sem1, sem2):
    cid = jax.lax.axis_index("c")
    sid = jax.lax.axis_index("s")
    wid = sid * 2 + cid
    base = SC_START + wid * SC_PER_W

    for j in range(C // 16):
        acc[pl.ds(16 * j, 16)] = jnp.zeros((16,), jnp.float32)

    bufs = (buf0, buf1, buf2)
    sems = (sem0, sem1, sem2)
    handles = [pltpu.async_copy(
        v2_hbm.at[pl.ds(base + k * CHUNK, CHUNK)], bufs[k], sems[k])
        for k in range(NCHUNK)]
    for k in range(NCHUNK):
        handles[k].wait()
        buf = bufs[k]

        for jb in range(C // 16 // 8):
            accs = tuple(acc[pl.ds(16 * (8 * jb + j), 16)] for j in range(8))

            def rbody(rr, accs, buf=buf, jb=jb):
                for u in range(RUNROLL):
                    r = rr * RUNROLL + u
                    accs = tuple(
                        a + jnp.abs(buf[r, pl.ds(16 * (8 * jb + j), 16)])
                        for j, a in enumerate(accs))
                return accs

            accs = jax.lax.fori_loop(0, CHUNK // RUNROLL, rbody, accs)
            for j in range(8):
                acc[pl.ds(16 * (8 * jb + j), 16)] = accs[j]

    pltpu.sync_copy(acc, out_hbm.at[wid])


_sc_partial = functools.partial(
    pl.kernel,
    out_type=jax.ShapeDtypeStruct((NW, C), jnp.float32),
    mesh=plsc.VectorSubcoreMesh(core_axis_name="c", subcore_axis_name="s"),
    scratch_types=[
        pltpu.VMEM((CHUNK, C), jnp.float32),
        pltpu.VMEM((CHUNK, C), jnp.float32),
        pltpu.VMEM((CHUNK, C), jnp.float32),
        pltpu.VMEM((C,), jnp.float32),
        pltpu.SemaphoreType.DMA,
        pltpu.SemaphoreType.DMA,
        pltpu.SemaphoreType.DMA,
    ],
)(_sc_body)


def _reduce_body(v1_ref, v2_ref, imp1_ref, cs2_ref, acc_ref):
    i = pl.program_id(0)
    half = (i % 2) * R

    rows = jnp.sum(jnp.abs(v1_ref[...]), axis=1, keepdims=True)

    @pl.when(i < 2)
    def _():
        acc_ref[pl.ds(half, R), :] = rows

    @pl.when(i >= 2)
    def _():
        acc_ref[pl.ds(half, R), :] = acc_ref[pl.ds(half, R), :] + rows

    colpart = jnp.sum(jnp.abs(v2_ref[...]), axis=0, keepdims=True)

    @pl.when(i == 0)
    def _():
        cs2_ref[...] = colpart

    @pl.when(i > 0)
    def _():
        cs2_ref[...] = cs2_ref[...] + colpart

    @pl.when(i == STEPS - 1)
    def _():
        imp1_ref[...] = acc_ref[...]


def _select_body(imp1_ref, cs2_ref, scp_ref, out_ref):
    imp2_row = cs2_ref[...] + jnp.sum(scp_ref[...], axis=0, keepdims=True)
    imp1_col = imp1_ref[...]
    eye = (jax.lax.broadcasted_iota(jnp.int32, (C, C), 0)
           == jax.lax.broadcasted_iota(jnp.int32, (C, C), 1)
           ).astype(jnp.float32)
    imp1_row = jax.lax.dot_general(
        imp1_col, eye, (((0,), (0,)), ((), ())), precision=_HI)
    imp2_col = jax.lax.dot_general(
        eye, imp2_row, (((1,), (1,)), ((), ())), precision=_HI)
    imp_col = imp1_col + imp2_col
    imp_row = imp1_row + imp2_row

    src_i = jax.lax.broadcasted_iota(jnp.int32, (C, C), 1)
    tgt_i = jax.lax.broadcasted_iota(jnp.int32, (C, C), 0)
    sel = (imp_row < imp_col) | ((imp_row == imp_col) & (src_i < tgt_i))
    rank = jnp.sum(jnp.where(sel, 1.0, 0.0), axis=1, keepdims=True)
    maskf = jnp.where(rank < float(SIZE), 1.0, 0.0)

    lower = jnp.where(src_i < tgt_i, 1.0, 0.0).astype(jnp.bfloat16)
    pos = jax.lax.dot_general(
        lower, maskf.astype(jnp.bfloat16), (((1,), (0,)), ((), ())),
        preferred_element_type=jnp.float32)

    slot = jax.lax.broadcasted_iota(jnp.int32, (C, SIZE), 1).astype(jnp.float32)
    w = maskf * jnp.where(pos == slot, 1.0, 0.0)
    lin = (-1.0 + jax.lax.broadcasted_iota(jnp.int32, (C, 1), 0)
           .astype(jnp.float32) * (2.0 / float(C - 1)))
    out_ref[...] = jnp.sum(w * lin, axis=0, keepdims=True)


@jax.jit
def _run(w1, w2):
    v1 = jnp.transpose(w1, (2, 3, 0, 1)).reshape(D, C)
    v2 = jnp.transpose(w2, (2, 3, 0, 1)).reshape(D, C)

    scpart = _sc_partial(v2)

    imp1, cs2 = pl.pallas_call(
        _reduce_body,
        grid=(STEPS,),
        in_specs=[
            pl.BlockSpec((R, C), lambda i: (i, 0)),
            pl.BlockSpec((R2, C), lambda i: (i, 0)),
        ],
        out_specs=[
            pl.BlockSpec((C, 1), lambda i: (0, 0)),
            pl.BlockSpec((1, C), lambda i: (0, 0)),
        ],
        out_shape=[
            jax.ShapeDtypeStruct((C, 1), jnp.float32),
            jax.ShapeDtypeStruct((1, C), jnp.float32),
        ],
        scratch_shapes=[pltpu.VMEM((C, 1), jnp.float32)],
        compiler_params=pltpu.CompilerParams(
            dimension_semantics=("arbitrary",),
        ),
    )(v1, v2)

    out = pl.pallas_call(
        _select_body,
        out_shape=jax.ShapeDtypeStruct((1, SIZE), jnp.float32),
    )(imp1, cs2, scpart)
    return out


def kernel(w1, w2, size):
    return _run(w1, w2).reshape(SIZE) + size * 0
